# baseline (speedup 1.0000x reference)
import functools

import jax
import jax.numpy as jnp
from jax import lax
from jax.experimental import pallas as pl
from jax.experimental.pallas import tpu as pltpu
from jax.experimental.pallas import tpu_sc as plsc

F32 = jnp.float32
CHUNK = 128
NW = 32


@functools.lru_cache(None)
def _sc_scatter(S, Np, n_chunks, nacc):
  stripe = nacc // 16
  gh = (n_chunks + 1) // 2
  mesh = plsc.VectorSubcoreMesh(core_axis_name="c", subcore_axis_name="s")
  nbuf = 2

  @functools.partial(
      pl.kernel, mesh=mesh,
      out_type=jax.ShapeDtypeStruct((2 * S * Np, 128), F32),
      scratch_types=[
          pltpu.VMEM((gh, CHUNK), jnp.int32),
          pltpu.VMEM((n_chunks, CHUNK), jnp.int32),
          pltpu.VMEM((nbuf, CHUNK, 128), F32),
          pltpu.VMEM_SHARED((nacc, 128), F32),
          pltpu.SemaphoreType.DMA,
          pltpu.SemaphoreType.DMA,
      ],
  )
  def body(u_hbm, gidx_hbm, sidx_hbm, out_hbm, gi_v, si_v, rows_v,
           acc_sh, gsem, ssem):
    cid = lax.axis_index("c")
    sid = lax.axis_index("s")
    wid = cid * 16 + sid

    def zrow(i, carry):
      for j in range(8):
        rows_v[0, i, pl.ds(j * 16, 16)] = jnp.zeros((16,), F32)
      return carry

    def fire_gather(k):
      pltpu.async_copy(u_hbm.at[gi_v.at[lax.rem(k, gh)]],
                       rows_v.at[lax.rem(k, nbuf)], gsem)

    def wait_gather(k):
      pltpu.make_async_copy(u_hbm.at[gi_v.at[lax.rem(k, gh)]],
                            rows_v.at[lax.rem(k, nbuf)], gsem).wait()

    def fire_scatter(k):
      pltpu.async_copy(rows_v.at[lax.rem(k, nbuf)],
                       acc_sh.at[si_v.at[k]], ssem, add=True)

    def wait_scatter(k):
      pltpu.make_async_copy(rows_v.at[lax.rem(k, nbuf)],
                            acc_sh.at[si_v.at[k]], ssem).wait()

    pltpu.sync_copy(sidx_hbm.at[wid], si_v)
    for s in range(S):
      pltpu.sync_copy(gidx_hbm.at[s, wid, pl.ds(0, gh)], gi_v)
      lax.fori_loop(0, CHUNK, zrow, 0)
      off = 0
      while off < stripe:
        sz = min(CHUNK, stripe - off)
        pltpu.sync_copy(rows_v.at[0, pl.ds(0, sz)],
                        acc_sh.at[pl.ds(sid * stripe + off, sz)])
        off += sz
      plsc.subcore_barrier()

      npairs = n_chunks // 2
      fire_gather(0)
      fire_gather(1)

      def pair(j, carry):
        k = j * 2
        wait_gather(k)
        wait_gather(k + 1)
        fire_scatter(k)
        fire_scatter(k + 1)
        wait_scatter(k)
        wait_scatter(k + 1)

        @pl.when(k + 2 == ((gh + 1) // 2) * 2)
        def _():
          pltpu.sync_copy(gidx_hbm.at[s, wid, pl.ds(gh, n_chunks - gh)],
                          gi_v.at[pl.ds(0, n_chunks - gh)])

        @pl.when(k + 3 < n_chunks)
        def _():
          fire_gather(k + 2)
          fire_gather(k + 3)
        return carry
      lax.fori_loop(0, npairs, pair, 0)
      if n_chunks % 2:
        k = n_chunks - 1
        fire_gather(k)
        wait_gather(k)
        fire_scatter(k)
        wait_scatter(k)
      plsc.subcore_barrier()

      ro = (cid * S + s) * Np + sid * stripe
      pltpu.sync_copy(acc_sh.at[pl.ds(sid * stripe, stripe)],
                      out_hbm.at[pl.ds(ro, stripe)])
      plsc.subcore_barrier()

  return body


def _rowmask(g, n_valid):
  rid = g * 128 + lax.broadcasted_iota(jnp.int32, (128, 1), 0)
  return rid < n_valid


@functools.lru_cache(None)
def _tc_dis(NB):
  def body(degp_ref, dis_ref):
    d = degp_ref[0, :, 0:1] + degp_ref[1, :, 0:1]
    dis = jnp.where(d > 0, lax.rsqrt(jnp.maximum(d, 1e-12)), 0.0)
    dis_ref[...] = jnp.broadcast_to(dis, (128, 128))

  return pl.pallas_call(
      body,
      grid=(NB,),
      in_specs=[pl.BlockSpec((2, 128, 128), lambda g: (0, g, 0))],
      out_specs=pl.BlockSpec((128, 128), lambda g: (g, 0)),
      out_shape=jax.ShapeDtypeStruct((NB * 128, 128), F32),
  )


@functools.lru_cache(None)
def _tc_prep(NB):
  def body(x_ref, dis_ref, u_ref):
    u_ref[0] = dis_ref[...] * x_ref[...]

  return pl.pallas_call(
      body,
      grid=(NB,),
      in_specs=[pl.BlockSpec((128, 128), lambda g: (g, 0)),
                pl.BlockSpec((128, 128), lambda g: (g, 0))],
      out_specs=pl.BlockSpec((1, 128, 128), lambda g: (0, g, 0)),
      out_shape=jax.ShapeDtypeStruct((1, NB * 128, 128), F32),
  )


@functools.lru_cache(None)
def _tc_prep_partial(NB, S):
  def body(p_ref, dis_ref, u_ref):
    dis = dis_ref[...]
    nd2 = -(dis * dis)
    for s in range(S):
      u_ref[s] = nd2 * (p_ref[0, s] + p_ref[1, s])

  return pl.pallas_call(
      body,
      grid=(NB,),
      in_specs=[pl.BlockSpec((2, S, 128, 128), lambda g: (0, 0, g, 0)),
                pl.BlockSpec((128, 128), lambda g: (g, 0))],
      out_specs=pl.BlockSpec((S, 128, 128), lambda g: (0, g, 0)),
      out_shape=jax.ShapeDtypeStruct((S, NB * 128, 128), F32),
  )


@functools.lru_cache(None)
def _tc_cheb(NB, n_valid, S, ci, co, K, with_stats):
  def body(*refs):
    if K == 3:
      h_ref, dis_ref, p1_ref, p2_ref, w_ref = refs[:5]
      outs = refs[5:]
    else:
      h_ref, dis_ref, p1_ref, w_ref = refs[:4]
      p2_ref = None
      outs = refs[4:]
    g = pl.program_id(0)
    dis = dis_ref[...]
    tx0 = h_ref[...]
    acc = jnp.dot(tx0, w_ref[0], preferred_element_type=F32)
    tx1 = jnp.concatenate(
        [-dis * (p1_ref[0, s] + p1_ref[1, s]) for s in range(S)], axis=1)
    acc = acc + jnp.dot(tx1, w_ref[1], preferred_element_type=F32)
    if K == 3:
      tx2 = jnp.concatenate(
          [(-2.0 * dis) * (p2_ref[0, s] + p2_ref[1, s]) for s in range(S)],
          axis=1) - tx0
      acc = acc + jnp.dot(tx2, w_ref[2], preferred_element_type=F32)
    outs[0][...] = acc
    if with_stats:
      m = _rowmask(g, n_valid)
      av = jnp.where(m, acc, 0.0)
      s1 = jnp.broadcast_to(jnp.sum(av, 0, keepdims=True), (8, co))
      s2 = jnp.broadcast_to(jnp.sum(av * av, 0, keepdims=True), (8, co))
      s1_ref, s2_ref = outs[1], outs[2]

      @pl.when(g == 0)
      def _():
        s1_ref[...] = s1
        s2_ref[...] = s2

      @pl.when(g != 0)
      def _():
        s1_ref[...] += s1
        s2_ref[...] += s2

  in_specs = [pl.BlockSpec((128, ci), lambda g: (g, 0)),
              pl.BlockSpec((128, 128), lambda g: (g, 0)),
              pl.BlockSpec((2, S, 128, 128), lambda g: (0, 0, g, 0))]
  if K == 3:
    in_specs.append(pl.BlockSpec((2, S, 128, 128), lambda g: (0, 0, g, 0)))
  in_specs.append(pl.BlockSpec((K, ci, co), lambda g: (0, 0, 0)))
  out_shapes = [jax.ShapeDtypeStruct((NB * 128, co), F32)]
  out_specs = [pl.BlockSpec((128, co), lambda g: (g, 0))]
  if with_stats:
    out_shapes += [jax.ShapeDtypeStruct((8, co), F32)] * 2
    out_specs += [pl.BlockSpec((8, co), lambda g: (0, 0))] * 2

  return pl.pallas_call(
      body, grid=(NB,), in_specs=in_specs, out_specs=out_specs,
      out_shape=out_shapes)


@functools.lru_cache(None)
def _tc_norm(NB, n_valid, S_next, co):
  nf = float(n_valid)

  def body(raw_ref, s1_ref, s2_ref, gw_ref, gb_ref, ms_ref, b_ref, dis_ref,
           h_ref, u_ref):
    b = b_ref[0:1, :]
    o = raw_ref[...] + b
    s1 = s1_ref[0:1, :]
    s2 = s2_ref[0:1, :]
    e1 = s1 / nf + b
    eo2 = (s2 + 2.0 * b * s1) / nf + b * b
    ms = ms_ref[0:1, :]
    var = eo2 - e1 * e1 * ms * (2.0 - ms)
    inv = lax.rsqrt(var + 1e-5)
    normed = gw_ref[0:1, :] * (o - e1 * ms) * inv + gb_ref[0:1, :]
    h = jnp.maximum(normed, 0.2 * normed)
    h_ref[...] = h
    dis = dis_ref[...]
    for s in range(S_next):
      u_ref[s] = dis * h[:, s * 128:(s + 1) * 128]

  def body_no_u(raw_ref, s1_ref, s2_ref, gw_ref, gb_ref, ms_ref, b_ref,
                dis_ref, h_ref):
    body(raw_ref, s1_ref, s2_ref, gw_ref, gb_ref, ms_ref, b_ref, dis_ref,
         h_ref, None)

  fixed = pl.BlockSpec((8, co), lambda g: (0, 0))
  out_specs = [pl.BlockSpec((128, co), lambda g: (g, 0))]
  out_shapes = [jax.ShapeDtypeStruct((NB * 128, co), F32)]
  if S_next:
    out_specs.append(pl.BlockSpec((S_next, 128, 128), lambda g: (0, g, 0)))
    out_shapes.append(jax.ShapeDtypeStruct((S_next, NB * 128, 128), F32))
  return pl.pallas_call(
      body if S_next else body_no_u,
      grid=(NB,),
      in_specs=[pl.BlockSpec((128, co), lambda g: (g, 0)),
                fixed, fixed, fixed, fixed, fixed, fixed,
                pl.BlockSpec((128, 128), lambda g: (g, 0))],
      out_specs=out_specs,
      out_shape=out_shapes,
  )


@functools.lru_cache(None)
def _tc_conv3a(NB):
  def body(h_ref, dis_ref, w_ref, base_ref, u_ref):
    h = h_ref[...]
    dis = dis_ref[...]
    hw0 = jnp.dot(h, w_ref[0], preferred_element_type=F32)
    hw1 = jnp.dot(h, w_ref[1], preferred_element_type=F32)
    hw2 = jnp.dot(h, w_ref[2], preferred_element_type=F32)
    base_ref[...] = hw0 - hw2
    for s in range(2):
      u_ref[s] = dis * hw1[:, s * 128:(s + 1) * 128]
      u_ref[2 + s] = dis * hw2[:, s * 128:(s + 1) * 128]

  return pl.pallas_call(
      body,
      grid=(NB,),
      in_specs=[pl.BlockSpec((128, 512), lambda g: (g, 0)),
                pl.BlockSpec((128, 128), lambda g: (g, 0)),
                pl.BlockSpec((3, 512, 256), lambda g: (0, 0, 0))],
      out_specs=[pl.BlockSpec((128, 256), lambda g: (g, 0)),
                 pl.BlockSpec((4, 128, 128), lambda g: (0, g, 0))],
      out_shape=[jax.ShapeDtypeStruct((NB * 128, 256), F32),
                 jax.ShapeDtypeStruct((4, NB * 128, 128), F32)],
  )


@functools.lru_cache(None)
def _tc_conv3b(NB, n_valid):
  def body(base_ref, dis_ref, p_ref, q_ref, raw_ref, s1_ref, s2_ref):
    g = pl.program_id(0)
    dis = dis_ref[...]
    t1 = jnp.concatenate(
        [-dis * (p_ref[0, s] + p_ref[1, s]) for s in range(2)], axis=1)
    t2 = jnp.concatenate(
        [(-2.0 * dis) * (q_ref[0, s] + q_ref[1, s]) for s in range(2)],
        axis=1)
    acc = base_ref[...] + t1 + t2
    raw_ref[...] = acc
    m = _rowmask(g, n_valid)
    av = jnp.where(m, acc, 0.0)
    s1 = jnp.broadcast_to(jnp.sum(av, 0, keepdims=True), (8, 256))
    s2 = jnp.broadcast_to(jnp.sum(av * av, 0, keepdims=True), (8, 256))

    @pl.when(g == 0)
    def _():
      s1_ref[...] = s1
      s2_ref[...] = s2

    @pl.when(g != 0)
    def _():
      s1_ref[...] += s1
      s2_ref[...] += s2

  return pl.pallas_call(
      body,
      grid=(NB,),
      in_specs=[pl.BlockSpec((128, 256), lambda g: (g, 0)),
                pl.BlockSpec((128, 128), lambda g: (g, 0)),
                pl.BlockSpec((2, 2, 128, 128), lambda g: (0, 0, g, 0)),
                pl.BlockSpec((2, 2, 128, 128), lambda g: (0, 0, g, 0))],
      out_specs=[pl.BlockSpec((128, 256), lambda g: (g, 0)),
                 pl.BlockSpec((8, 256), lambda g: (0, 0)),
                 pl.BlockSpec((8, 256), lambda g: (0, 0))],
      out_shape=[jax.ShapeDtypeStruct((NB * 128, 256), F32),
                 jax.ShapeDtypeStruct((8, 256), F32),
                 jax.ShapeDtypeStruct((8, 256), F32)],
  )


@functools.lru_cache(None)
def _tc_conv4a(NB, ci):
  def body(h_ref, dis_ref, w_ref, hw0_ref, u_ref):
    h = h_ref[...]
    hw0_ref[...] = jnp.dot(h, w_ref[0], preferred_element_type=F32)
    u_ref[0] = dis_ref[...] * jnp.dot(h, w_ref[1], preferred_element_type=F32)

  return pl.pallas_call(
      body,
      grid=(NB,),
      in_specs=[pl.BlockSpec((128, ci), lambda g: (g, 0)),
                pl.BlockSpec((128, 128), lambda g: (g, 0)),
                pl.BlockSpec((2, ci, 128), lambda g: (0, 0, 0))],
      out_specs=[pl.BlockSpec((128, 128), lambda g: (g, 0)),
                 pl.BlockSpec((1, 128, 128), lambda g: (0, g, 0))],
      out_shape=[jax.ShapeDtypeStruct((NB * 128, 128), F32),
                 jax.ShapeDtypeStruct((1, NB * 128, 128), F32)],
  )


@functools.lru_cache(None)
def _tc_resid(NB):
  def body(hw0_ref, p_ref, b_ref, x_ref, dis_ref, cur_ref, u_ref):
    dis = dis_ref[...]
    o = (hw0_ref[...] - dis * (p_ref[0, 0] + p_ref[1, 0])
         + b_ref[0:1, :] + x_ref[...])
    cur = jnp.maximum(o, 0.0)
    cur_ref[...] = cur
    u_ref[0] = dis * cur

  blk = pl.BlockSpec((128, 128), lambda g: (g, 0))
  return pl.pallas_call(
      body,
      grid=(NB,),
      in_specs=[blk,
                pl.BlockSpec((2, 1, 128, 128), lambda g: (0, 0, g, 0)),
                pl.BlockSpec((8, 128), lambda g: (0, 0)), blk, blk],
      out_specs=[blk, pl.BlockSpec((1, 128, 128), lambda g: (0, g, 0))],
      out_shape=[jax.ShapeDtypeStruct((NB * 128, 128), F32),
                 jax.ShapeDtypeStruct((1, NB * 128, 128), F32)],
  )


@functools.lru_cache(None)
def _tc_pool(NB, n_valid):
  def body(cur_ref, acc_ref):
    g = pl.program_id(0)
    m = _rowmask(g, n_valid)
    s = jnp.broadcast_to(
        jnp.sum(jnp.where(m, cur_ref[...], 0.0), 0, keepdims=True), (8, 128))

    @pl.when(g == 0)
    def _():
      acc_ref[...] = s

    @pl.when(g != 0)
    def _():
      acc_ref[...] += s

  return pl.pallas_call(
      body,
      grid=(NB,),
      in_specs=[pl.BlockSpec((128, 128), lambda g: (g, 0))],
      out_specs=pl.BlockSpec((8, 128), lambda g: (0, 0)),
      out_shape=jax.ShapeDtypeStruct((8, 128), F32),
  )


@functools.lru_cache(None)
def _tc_mlp(n_valid):
  nf = float(n_valid)

  def body(pooled_ref, w1_ref, b1_ref, w2_ref, b2_ref, out_ref):
    pm = pooled_ref[0:1, :] * (1.0 / nf)
    h1 = jnp.tanh(jnp.dot(pm, w1_ref[...], preferred_element_type=F32)
                  + b1_ref[0:1, :])
    o = jnp.dot(h1, w2_ref[...], preferred_element_type=F32) + b2_ref[0:1, :]
    out_ref[...] = jnp.broadcast_to(o, (8, 128))

  return pl.pallas_call(
      body,
      grid=(1,),
      in_specs=[pl.BlockSpec((8, 128), lambda g: (0, 0)),
                pl.BlockSpec((128, 128), lambda g: (0, 0)),
                pl.BlockSpec((8, 128), lambda g: (0, 0)),
                pl.BlockSpec((128, 128), lambda g: (0, 0)),
                pl.BlockSpec((8, 128), lambda g: (0, 0))],
      out_specs=pl.BlockSpec((8, 128), lambda g: (0, 0)),
      out_shape=jax.ShapeDtypeStruct((8, 128), F32),
  )


_CONV_SPECS = [(128, 256, 2), (256, 512, 3), (512, 256, 3), (256, 128, 2)]


def kernel(x, edge_index, batch, params):
  n, d = x.shape
  e = edge_index.shape[1]
  np_ = ((n + 128) // 128) * 128
  nb = np_ // 128
  ew = ((e + NW * CHUNK - 1) // (NW * CHUNK)) * CHUNK
  e_pad = ew * NW
  n_chunks = ew // CHUNK

  row = edge_index[0]
  col = edge_index[1]
  pad = jnp.full((e_pad - e,), n, jnp.int32)
  rp = jnp.concatenate([row, pad])
  cp = jnp.concatenate([col, pad])
  gidx = (rp[None, :]
          + (jnp.arange(4, dtype=jnp.int32) * np_)[:, None]
          ).reshape(4, NW, n_chunks, CHUNK)
  sidx = cp.reshape(NW, n_chunks, CHUNK)
  ridx = rp.reshape(NW, n_chunks, CHUNK)
  xp = jnp.pad(x, ((0, np_ - n), (0, 0)))

  def bc8(v):
    return jnp.broadcast_to(v[None, :], (8, v.shape[0]))

  nacc = np_
  ones_tab = jnp.ones((np_, 128), F32)
  degp = _sc_scatter(1, np_, n_chunks, nacc)(ones_tab, gidx[:1], ridx)
  dis = _tc_dis(nb)(degp.reshape(2, np_, 128))

  def lhat_partials(u3, S):
    u_flat = u3.reshape(S * np_, 128)
    p = _sc_scatter(S, np_, n_chunks, nacc)(u_flat, gidx[:S], sidx)
    return p.reshape(2, S, np_, 128)

  u = _tc_prep(nb)(xp, dis)
  cur = xp
  gi = 0

  def norm(raw, s1, s2, cw, s_next, co):
    nonlocal gi
    gn = params["gn"][gi]
    gi += 1
    return _tc_norm(nb, n, s_next, co)(
        raw, s1, s2, bc8(gn["weight"]), bc8(gn["bias"]),
        bc8(gn["mean_scale"]), bc8(cw["b"]), dis)

  for block in range(4):
    h = cur
    cw = params["conv"][block * 4]
    p1 = lhat_partials(u, 1)
    raw, s1, s2 = _tc_cheb(nb, n, 1, 128, 256, 2, True)(h, dis, p1, cw["W"])
    h, u = norm(raw, s1, s2, cw, 2, 256)
    cw = params["conv"][block * 4 + 1]
    p1 = lhat_partials(u, 2)
    u2 = _tc_prep_partial(nb, 2)(p1, dis)
    p2 = lhat_partials(u2, 2)
    raw, s1, s2 = _tc_cheb(nb, n, 2, 256, 512, 3, True)(
        h, dis, p1, p2, cw["W"])
    h = norm(raw, s1, s2, cw, 0, 512)[0]
    cw = params["conv"][block * 4 + 2]
    base, u12 = _tc_conv3a(nb)(h, dis, cw["W"])
    p = lhat_partials(u12, 4)
    d2 = _tc_prep_partial(nb, 2)(p[:, 2:4], dis)
    q = lhat_partials(d2, 2)
    raw, s1, s2 = _tc_conv3b(nb, n)(base, dis, p[:, 0:2], q)
    h = norm(raw, s1, s2, cw, 0, 256)[0]
    cw = params["conv"][block * 4 + 3]
    hw0, ua = _tc_conv4a(nb, 256)(h, dis, cw["W"])
    p1 = lhat_partials(ua, 1)
    cur, u = _tc_resid(nb)(hw0, p1, bc8(cw["b"]), xp, dis)

  pooled = _tc_pool(nb, n)(cur)
  w1 = jnp.zeros((128, 128), F32).at[:, :64].set(params["lin"][0]["W"])
  b1 = jnp.zeros((128,), F32).at[:64].set(params["lin"][0]["b"])
  w2 = jnp.zeros((128, 128), F32).at[:64, :10].set(params["lin"][1]["W"])
  b2 = jnp.zeros((128,), F32).at[:10].set(params["lin"][1]["b"])
  out = _tc_mlp(n)(pooled, w1, bc8(b1), w2, bc8(b2))
  return out[0:1, 0:10]


# speedup vs baseline: 1.1406x; 1.1406x over previous
import functools

import jax
import jax.numpy as jnp
from jax import lax
from jax.experimental import pallas as pl
from jax.experimental.pallas import tpu as pltpu
from jax.experimental.pallas import tpu_sc as plsc

F32 = jnp.float32
CHUNK = 128
NW = 32


@functools.lru_cache(None)
def _sc_scatter(S, Np, n_chunks, nacc):
  stripe = nacc // 16
  gh = (n_chunks + 1) // 2
  mesh = plsc.VectorSubcoreMesh(core_axis_name="c", subcore_axis_name="s")
  nbuf = 2

  @functools.partial(
      pl.kernel, mesh=mesh,
      out_type=jax.ShapeDtypeStruct((2 * S * Np, 128), F32),
      scratch_types=[
          pltpu.VMEM((gh, CHUNK), jnp.int32),
          pltpu.VMEM((n_chunks, CHUNK), jnp.int32),
          pltpu.VMEM((nbuf, CHUNK, 128), F32),
          pltpu.VMEM_SHARED((nacc, 128), F32),
          pltpu.SemaphoreType.DMA,
          pltpu.SemaphoreType.DMA,
      ],
  )
  def body(u_hbm, gidx_hbm, sidx_hbm, out_hbm, gi_v, si_v, rows_v,
           acc_sh, gsem, ssem):
    cid = lax.axis_index("c")
    sid = lax.axis_index("s")
    wid = cid * 16 + sid

    def zrow(i, carry):
      for j in range(8):
        rows_v[0, i, pl.ds(j * 16, 16)] = jnp.zeros((16,), F32)
      return carry

    def fire_gather(k):
      pltpu.async_copy(u_hbm.at[gi_v.at[lax.rem(k, gh)]],
                       rows_v.at[lax.rem(k, nbuf)], gsem)

    def wait_gather(k):
      pltpu.make_async_copy(u_hbm.at[gi_v.at[lax.rem(k, gh)]],
                            rows_v.at[lax.rem(k, nbuf)], gsem).wait()

    def fire_scatter(k):
      pltpu.async_copy(rows_v.at[lax.rem(k, nbuf)],
                       acc_sh.at[si_v.at[k]], ssem, add=True)

    def wait_scatter(k):
      pltpu.make_async_copy(rows_v.at[lax.rem(k, nbuf)],
                            acc_sh.at[si_v.at[k]], ssem).wait()

    pltpu.sync_copy(sidx_hbm.at[wid], si_v)
    for s in range(S):
      pltpu.sync_copy(gidx_hbm.at[s, wid, pl.ds(0, gh)], gi_v)
      lax.fori_loop(0, CHUNK, zrow, 0)
      off = 0
      while off < stripe:
        sz = min(CHUNK, stripe - off)
        pltpu.sync_copy(rows_v.at[0, pl.ds(0, sz)],
                        acc_sh.at[pl.ds(sid * stripe + off, sz)])
        off += sz
      plsc.subcore_barrier()

      for k in range(min(nbuf - 1, n_chunks)):
        fire_gather(k)

      def chunk(k, carry):
        @pl.when(k + nbuf - 1 < n_chunks)
        def _():
          @pl.when(k >= 1)
          def _():
            wait_scatter(k - 1)

          @pl.when(k + nbuf - 1 == gh)
          def _():
            pltpu.sync_copy(gidx_hbm.at[s, wid, pl.ds(gh, n_chunks - gh)],
                            gi_v.at[pl.ds(0, n_chunks - gh)])
          fire_gather(k + nbuf - 1)
        wait_gather(k)
        fire_scatter(k)
        return carry
      lax.fori_loop(0, n_chunks, chunk, 0)
      for k in range(max(n_chunks - nbuf + 1, 1) - 1, n_chunks):
        wait_scatter(k)
      plsc.subcore_barrier()

      ro = (cid * S + s) * Np + sid * stripe
      pltpu.sync_copy(acc_sh.at[pl.ds(sid * stripe, stripe)],
                      out_hbm.at[pl.ds(ro, stripe)])
      plsc.subcore_barrier()

  return body


def _rowmask(g, n_valid):
  rid = g * 128 + lax.broadcasted_iota(jnp.int32, (128, 1), 0)
  return rid < n_valid


@functools.lru_cache(None)
def _tc_dis(NB):
  def body(degp_ref, dis_ref):
    d = degp_ref[0, :, 0:1] + degp_ref[1, :, 0:1]
    dis = jnp.where(d > 0, lax.rsqrt(jnp.maximum(d, 1e-12)), 0.0)
    dis_ref[...] = jnp.broadcast_to(dis, (128, 128))

  return pl.pallas_call(
      body,
      grid=(NB,),
      in_specs=[pl.BlockSpec((2, 128, 128), lambda g: (0, g, 0))],
      out_specs=pl.BlockSpec((128, 128), lambda g: (g, 0)),
      out_shape=jax.ShapeDtypeStruct((NB * 128, 128), F32),
  )


@functools.lru_cache(None)
def _tc_prep(NB):
  def body(x_ref, dis_ref, u_ref):
    u_ref[0] = dis_ref[...] * x_ref[...]

  return pl.pallas_call(
      body,
      grid=(NB,),
      in_specs=[pl.BlockSpec((128, 128), lambda g: (g, 0)),
                pl.BlockSpec((128, 128), lambda g: (g, 0))],
      out_specs=pl.BlockSpec((1, 128, 128), lambda g: (0, g, 0)),
      out_shape=jax.ShapeDtypeStruct((1, NB * 128, 128), F32),
  )


@functools.lru_cache(None)
def _tc_prep_partial(NB, S):
  def body(p_ref, dis_ref, u_ref):
    dis = dis_ref[...]
    nd2 = -(dis * dis)
    for s in range(S):
      u_ref[s] = nd2 * (p_ref[0, s] + p_ref[1, s])

  return pl.pallas_call(
      body,
      grid=(NB,),
      in_specs=[pl.BlockSpec((2, S, 128, 128), lambda g: (0, 0, g, 0)),
                pl.BlockSpec((128, 128), lambda g: (g, 0))],
      out_specs=pl.BlockSpec((S, 128, 128), lambda g: (0, g, 0)),
      out_shape=jax.ShapeDtypeStruct((S, NB * 128, 128), F32),
  )


@functools.lru_cache(None)
def _tc_cheb(NB, n_valid, S, ci, co, K, with_stats):
  def body(*refs):
    if K == 3:
      h_ref, dis_ref, p1_ref, p2_ref, w_ref = refs[:5]
      outs = refs[5:]
    else:
      h_ref, dis_ref, p1_ref, w_ref = refs[:4]
      p2_ref = None
      outs = refs[4:]
    g = pl.program_id(0)
    dis = dis_ref[...]
    tx0 = h_ref[...]
    acc = jnp.dot(tx0, w_ref[0], preferred_element_type=F32)
    tx1 = jnp.concatenate(
        [-dis * (p1_ref[0, s] + p1_ref[1, s]) for s in range(S)], axis=1)
    acc = acc + jnp.dot(tx1, w_ref[1], preferred_element_type=F32)
    if K == 3:
      tx2 = jnp.concatenate(
          [(-2.0 * dis) * (p2_ref[0, s] + p2_ref[1, s]) for s in range(S)],
          axis=1) - tx0
      acc = acc + jnp.dot(tx2, w_ref[2], preferred_element_type=F32)
    outs[0][...] = acc
    if with_stats:
      m = _rowmask(g, n_valid)
      av = jnp.where(m, acc, 0.0)
      s1 = jnp.broadcast_to(jnp.sum(av, 0, keepdims=True), (8, co))
      s2 = jnp.broadcast_to(jnp.sum(av * av, 0, keepdims=True), (8, co))
      s1_ref, s2_ref = outs[1], outs[2]

      @pl.when(g == 0)
      def _():
        s1_ref[...] = s1
        s2_ref[...] = s2

      @pl.when(g != 0)
      def _():
        s1_ref[...] += s1
        s2_ref[...] += s2

  in_specs = [pl.BlockSpec((128, ci), lambda g: (g, 0)),
              pl.BlockSpec((128, 128), lambda g: (g, 0)),
              pl.BlockSpec((2, S, 128, 128), lambda g: (0, 0, g, 0))]
  if K == 3:
    in_specs.append(pl.BlockSpec((2, S, 128, 128), lambda g: (0, 0, g, 0)))
  in_specs.append(pl.BlockSpec((K, ci, co), lambda g: (0, 0, 0)))
  out_shapes = [jax.ShapeDtypeStruct((NB * 128, co), F32)]
  out_specs = [pl.BlockSpec((128, co), lambda g: (g, 0))]
  if with_stats:
    out_shapes += [jax.ShapeDtypeStruct((8, co), F32)] * 2
    out_specs += [pl.BlockSpec((8, co), lambda g: (0, 0))] * 2

  return pl.pallas_call(
      body, grid=(NB,), in_specs=in_specs, out_specs=out_specs,
      out_shape=out_shapes)


@functools.lru_cache(None)
def _tc_norm(NB, n_valid, S_next, co):
  nf = float(n_valid)

  def body(raw_ref, s1_ref, s2_ref, gw_ref, gb_ref, ms_ref, b_ref, dis_ref,
           h_ref, u_ref):
    b = b_ref[0:1, :]
    o = raw_ref[...] + b
    s1 = s1_ref[0:1, :]
    s2 = s2_ref[0:1, :]
    e1 = s1 / nf + b
    eo2 = (s2 + 2.0 * b * s1) / nf + b * b
    ms = ms_ref[0:1, :]
    var = eo2 - e1 * e1 * ms * (2.0 - ms)
    inv = lax.rsqrt(var + 1e-5)
    normed = gw_ref[0:1, :] * (o - e1 * ms) * inv + gb_ref[0:1, :]
    h = jnp.maximum(normed, 0.2 * normed)
    h_ref[...] = h
    dis = dis_ref[...]
    for s in range(S_next):
      u_ref[s] = dis * h[:, s * 128:(s + 1) * 128]

  def body_no_u(raw_ref, s1_ref, s2_ref, gw_ref, gb_ref, ms_ref, b_ref,
                dis_ref, h_ref):
    body(raw_ref, s1_ref, s2_ref, gw_ref, gb_ref, ms_ref, b_ref, dis_ref,
         h_ref, None)

  fixed = pl.BlockSpec((8, co), lambda g: (0, 0))
  out_specs = [pl.BlockSpec((128, co), lambda g: (g, 0))]
  out_shapes = [jax.ShapeDtypeStruct((NB * 128, co), F32)]
  if S_next:
    out_specs.append(pl.BlockSpec((S_next, 128, 128), lambda g: (0, g, 0)))
    out_shapes.append(jax.ShapeDtypeStruct((S_next, NB * 128, 128), F32))
  return pl.pallas_call(
      body if S_next else body_no_u,
      grid=(NB,),
      in_specs=[pl.BlockSpec((128, co), lambda g: (g, 0)),
                fixed, fixed, fixed, fixed, fixed, fixed,
                pl.BlockSpec((128, 128), lambda g: (g, 0))],
      out_specs=out_specs,
      out_shape=out_shapes,
  )


@functools.lru_cache(None)
def _tc_conv3a(NB):
  def body(h_ref, dis_ref, w_ref, base_ref, u_ref):
    h = h_ref[...]
    dis = dis_ref[...]
    hw0 = jnp.dot(h, w_ref[0], preferred_element_type=F32)
    hw1 = jnp.dot(h, w_ref[1], preferred_element_type=F32)
    hw2 = jnp.dot(h, w_ref[2], preferred_element_type=F32)
    base_ref[...] = hw0 - hw2
    for s in range(2):
      u_ref[s] = dis * hw1[:, s * 128:(s + 1) * 128]
      u_ref[2 + s] = dis * hw2[:, s * 128:(s + 1) * 128]

  return pl.pallas_call(
      body,
      grid=(NB,),
      in_specs=[pl.BlockSpec((128, 512), lambda g: (g, 0)),
                pl.BlockSpec((128, 128), lambda g: (g, 0)),
                pl.BlockSpec((3, 512, 256), lambda g: (0, 0, 0))],
      out_specs=[pl.BlockSpec((128, 256), lambda g: (g, 0)),
                 pl.BlockSpec((4, 128, 128), lambda g: (0, g, 0))],
      out_shape=[jax.ShapeDtypeStruct((NB * 128, 256), F32),
                 jax.ShapeDtypeStruct((4, NB * 128, 128), F32)],
  )


@functools.lru_cache(None)
def _tc_conv3b(NB, n_valid):
  def body(base_ref, dis_ref, p_ref, q_ref, raw_ref, s1_ref, s2_ref):
    g = pl.program_id(0)
    dis = dis_ref[...]
    t1 = jnp.concatenate(
        [-dis * (p_ref[0, s] + p_ref[1, s]) for s in range(2)], axis=1)
    t2 = jnp.concatenate(
        [(-2.0 * dis) * (q_ref[0, s] + q_ref[1, s]) for s in range(2)],
        axis=1)
    acc = base_ref[...] + t1 + t2
    raw_ref[...] = acc
    m = _rowmask(g, n_valid)
    av = jnp.where(m, acc, 0.0)
    s1 = jnp.broadcast_to(jnp.sum(av, 0, keepdims=True), (8, 256))
    s2 = jnp.broadcast_to(jnp.sum(av * av, 0, keepdims=True), (8, 256))

    @pl.when(g == 0)
    def _():
      s1_ref[...] = s1
      s2_ref[...] = s2

    @pl.when(g != 0)
    def _():
      s1_ref[...] += s1
      s2_ref[...] += s2

  return pl.pallas_call(
      body,
      grid=(NB,),
      in_specs=[pl.BlockSpec((128, 256), lambda g: (g, 0)),
                pl.BlockSpec((128, 128), lambda g: (g, 0)),
                pl.BlockSpec((2, 2, 128, 128), lambda g: (0, 0, g, 0)),
                pl.BlockSpec((2, 2, 128, 128), lambda g: (0, 0, g, 0))],
      out_specs=[pl.BlockSpec((128, 256), lambda g: (g, 0)),
                 pl.BlockSpec((8, 256), lambda g: (0, 0)),
                 pl.BlockSpec((8, 256), lambda g: (0, 0))],
      out_shape=[jax.ShapeDtypeStruct((NB * 128, 256), F32),
                 jax.ShapeDtypeStruct((8, 256), F32),
                 jax.ShapeDtypeStruct((8, 256), F32)],
  )


@functools.lru_cache(None)
def _tc_conv4a(NB, ci):
  def body(h_ref, dis_ref, w_ref, hw0_ref, u_ref):
    h = h_ref[...]
    hw0_ref[...] = jnp.dot(h, w_ref[0], preferred_element_type=F32)
    u_ref[0] = dis_ref[...] * jnp.dot(h, w_ref[1], preferred_element_type=F32)

  return pl.pallas_call(
      body,
      grid=(NB,),
      in_specs=[pl.BlockSpec((128, ci), lambda g: (g, 0)),
                pl.BlockSpec((128, 128), lambda g: (g, 0)),
                pl.BlockSpec((2, ci, 128), lambda g: (0, 0, 0))],
      out_specs=[pl.BlockSpec((128, 128), lambda g: (g, 0)),
                 pl.BlockSpec((1, 128, 128), lambda g: (0, g, 0))],
      out_shape=[jax.ShapeDtypeStruct((NB * 128, 128), F32),
                 jax.ShapeDtypeStruct((1, NB * 128, 128), F32)],
  )


@functools.lru_cache(None)
def _tc_resid(NB):
  def body(hw0_ref, p_ref, b_ref, x_ref, dis_ref, cur_ref, u_ref):
    dis = dis_ref[...]
    o = (hw0_ref[...] - dis * (p_ref[0, 0] + p_ref[1, 0])
         + b_ref[0:1, :] + x_ref[...])
    cur = jnp.maximum(o, 0.0)
    cur_ref[...] = cur
    u_ref[0] = dis * cur

  blk = pl.BlockSpec((128, 128), lambda g: (g, 0))
  return pl.pallas_call(
      body,
      grid=(NB,),
      in_specs=[blk,
                pl.BlockSpec((2, 1, 128, 128), lambda g: (0, 0, g, 0)),
                pl.BlockSpec((8, 128), lambda g: (0, 0)), blk, blk],
      out_specs=[blk, pl.BlockSpec((1, 128, 128), lambda g: (0, g, 0))],
      out_shape=[jax.ShapeDtypeStruct((NB * 128, 128), F32),
                 jax.ShapeDtypeStruct((1, NB * 128, 128), F32)],
  )


@functools.lru_cache(None)
def _tc_pool(NB, n_valid):
  def body(cur_ref, acc_ref):
    g = pl.program_id(0)
    m = _rowmask(g, n_valid)
    s = jnp.broadcast_to(
        jnp.sum(jnp.where(m, cur_ref[...], 0.0), 0, keepdims=True), (8, 128))

    @pl.when(g == 0)
    def _():
      acc_ref[...] = s

    @pl.when(g != 0)
    def _():
      acc_ref[...] += s

  return pl.pallas_call(
      body,
      grid=(NB,),
      in_specs=[pl.BlockSpec((128, 128), lambda g: (g, 0))],
      out_specs=pl.BlockSpec((8, 128), lambda g: (0, 0)),
      out_shape=jax.ShapeDtypeStruct((8, 128), F32),
  )


@functools.lru_cache(None)
def _tc_mlp(n_valid):
  nf = float(n_valid)

  def body(pooled_ref, w1_ref, b1_ref, w2_ref, b2_ref, out_ref):
    pm = pooled_ref[0:1, :] * (1.0 / nf)
    h1 = jnp.tanh(jnp.dot(pm, w1_ref[...], preferred_element_type=F32)
                  + b1_ref[0:1, :])
    o = jnp.dot(h1, w2_ref[...], preferred_element_type=F32) + b2_ref[0:1, :]
    out_ref[...] = jnp.broadcast_to(o, (8, 128))

  return pl.pallas_call(
      body,
      grid=(1,),
      in_specs=[pl.BlockSpec((8, 128), lambda g: (0, 0)),
                pl.BlockSpec((128, 128), lambda g: (0, 0)),
                pl.BlockSpec((8, 128), lambda g: (0, 0)),
                pl.BlockSpec((128, 128), lambda g: (0, 0)),
                pl.BlockSpec((8, 128), lambda g: (0, 0))],
      out_specs=pl.BlockSpec((8, 128), lambda g: (0, 0)),
      out_shape=jax.ShapeDtypeStruct((8, 128), F32),
  )


_CONV_SPECS = [(128, 256, 2), (256, 512, 3), (512, 256, 3), (256, 128, 2)]


def kernel(x, edge_index, batch, params):
  n, d = x.shape
  e = edge_index.shape[1]
  np_ = ((n + 128) // 128) * 128
  nb = np_ // 128
  ew = ((e + NW * CHUNK - 1) // (NW * CHUNK)) * CHUNK
  e_pad = ew * NW
  n_chunks = ew // CHUNK

  row = edge_index[0]
  col = edge_index[1]
  pad = jnp.full((e_pad - e,), n, jnp.int32)
  rp = jnp.concatenate([row, pad])
  cp = jnp.concatenate([col, pad])
  gidx = (rp[None, :]
          + (jnp.arange(4, dtype=jnp.int32) * np_)[:, None]
          ).reshape(4, NW, n_chunks, CHUNK)
  sidx = cp.reshape(NW, n_chunks, CHUNK)
  ridx = rp.reshape(NW, n_chunks, CHUNK)
  xp = jnp.pad(x, ((0, np_ - n), (0, 0)))

  def bc8(v):
    return jnp.broadcast_to(v[None, :], (8, v.shape[0]))

  nacc = np_
  ones_tab = jnp.ones((np_, 128), F32)
  degp = _sc_scatter(1, np_, n_chunks, nacc)(ones_tab, gidx[:1], ridx)
  dis = _tc_dis(nb)(degp.reshape(2, np_, 128))

  def lhat_partials(u3, S):
    u_flat = u3.reshape(S * np_, 128)
    p = _sc_scatter(S, np_, n_chunks, nacc)(u_flat, gidx[:S], sidx)
    return p.reshape(2, S, np_, 128)

  u = _tc_prep(nb)(xp, dis)
  cur = xp
  gi = 0

  def norm(raw, s1, s2, cw, s_next, co):
    nonlocal gi
    gn = params["gn"][gi]
    gi += 1
    return _tc_norm(nb, n, s_next, co)(
        raw, s1, s2, bc8(gn["weight"]), bc8(gn["bias"]),
        bc8(gn["mean_scale"]), bc8(cw["b"]), dis)

  for block in range(4):
    h = cur
    cw = params["conv"][block * 4]
    p1 = lhat_partials(u, 1)
    raw, s1, s2 = _tc_cheb(nb, n, 1, 128, 256, 2, True)(h, dis, p1, cw["W"])
    h, u = norm(raw, s1, s2, cw, 2, 256)
    cw = params["conv"][block * 4 + 1]
    p1 = lhat_partials(u, 2)
    u2 = _tc_prep_partial(nb, 2)(p1, dis)
    p2 = lhat_partials(u2, 2)
    raw, s1, s2 = _tc_cheb(nb, n, 2, 256, 512, 3, True)(
        h, dis, p1, p2, cw["W"])
    h = norm(raw, s1, s2, cw, 0, 512)[0]
    cw = params["conv"][block * 4 + 2]
    base, u12 = _tc_conv3a(nb)(h, dis, cw["W"])
    p = lhat_partials(u12, 4)
    d2 = _tc_prep_partial(nb, 2)(p[:, 2:4], dis)
    q = lhat_partials(d2, 2)
    raw, s1, s2 = _tc_conv3b(nb, n)(base, dis, p[:, 0:2], q)
    h = norm(raw, s1, s2, cw, 0, 256)[0]
    cw = params["conv"][block * 4 + 3]
    hw0, ua = _tc_conv4a(nb, 256)(h, dis, cw["W"])
    p1 = lhat_partials(ua, 1)
    cur, u = _tc_resid(nb)(hw0, p1, bc8(cw["b"]), xp, dis)

  pooled = _tc_pool(nb, n)(cur)
  w1 = jnp.zeros((128, 128), F32).at[:, :64].set(params["lin"][0]["W"])
  b1 = jnp.zeros((128,), F32).at[:64].set(params["lin"][0]["b"])
  w2 = jnp.zeros((128, 128), F32).at[:64, :10].set(params["lin"][1]["W"])
  b2 = jnp.zeros((128,), F32).at[:10].set(params["lin"][1]["b"])
  out = _tc_mlp(n)(pooled, w1, bc8(b1), w2, bc8(b2))
  return out[0:1, 0:10]
